# baseline (speedup 1.0000x reference)
import functools

import jax
import jax.numpy as jnp
from jax import lax
from jax.experimental import pallas as pl
from jax.experimental.pallas import tpu as pltpu
from jax.experimental.pallas import tpu_sc as plsc

R, C = 16384, 512
OC = C // 2
NW = 32
ROWS_PER_W = R // NW
N_CHUNK = 8
CH_ROWS = ROWS_PER_W // N_CHUNK
LANES = 16
CP = C + 1
OCP = OC + 8
ITERS = CH_ROWS * OC // LANES

_mesh = plsc.VectorSubcoreMesh(core_axis_name="c", subcore_axis_name="s")


@functools.partial(
    pl.kernel,
    mesh=_mesh,
    out_type=jax.ShapeDtypeStruct((R, OC), jnp.float32),
    scratch_types=[
        pltpu.VMEM((CH_ROWS, CP), jnp.float32),
        pltpu.VMEM((CH_ROWS, CP), jnp.float32),
        pltpu.VMEM((CH_ROWS, OCP), jnp.float32),
        pltpu.VMEM((CH_ROWS, OCP), jnp.float32),
        pltpu.SemaphoreType.DMA,
        pltpu.SemaphoreType.DMA,
    ],
    compiler_params=pltpu.CompilerParams(needs_layout_passes=False),
)
def _deinterleave(in_hbm, out_hbm, in_v0, in_v1, out_v0, out_v1,
                  in_sem, out_sem):
    wid = lax.axis_index("s") * 2 + lax.axis_index("c")
    row_base = wid * ROWS_PER_W
    lane = lax.iota(jnp.int32, LANES)
    step8 = (lane >> 3)
    lane7 = lane & 7
    cin0 = lane7 * 2
    in_bufs = (in_v0, in_v1)
    out_bufs = (out_v0, out_v1)

    def in_copy(c):
        return pltpu.async_copy(
            in_hbm.at[pl.ds(row_base + c * CH_ROWS, CH_ROWS), :],
            in_bufs[c % 2].at[:, pl.ds(0, C)], in_sem)

    def out_copy(c):
        return pltpu.async_copy(
            out_bufs[c % 2].at[:, pl.ds(0, OC)],
            out_hbm.at[pl.ds(row_base + c * CH_ROWS, CH_ROWS), :],
            out_sem)

    in_h = in_copy(0)
    out_h = [None, None]
    for c in range(N_CHUNK):
        in_h.wait()
        if c + 1 < N_CHUNK:
            in_h = in_copy(c + 1)
        if out_h[c % 2] is not None:
            out_h[c % 2].wait()
        iv = in_bufs[c % 2]
        ov = out_bufs[c % 2]

        @plsc.parallel_loop(0, ITERS, 1, unroll=8)
        def _(i):
            p = i >> 5
            j = i & 31
            row = p * 2 + step8
            g = plsc.load_gather(iv, [row, cin0 + j * LANES])
            plsc.store_scatter(ov, [row, lane7 + j * 8], g)

        out_h[c % 2] = out_copy(c)
    out_h[0].wait()
    out_h[1].wait()


def kernel(inputs):
    return _deinterleave(inputs)
